# baseline (speedup 1.0000x reference)
import functools

import jax
import jax.numpy as jnp
from jax import lax
from jax.experimental import pallas as pl
from jax.experimental.pallas import tpu as pltpu
from jax.experimental.pallas import tpu_sc as plsc

_NC = 2
_NS = 16


def _chunk_sizes(total_rows, max_rows):
    n = -(-total_rows // max_rows)
    sizes = []
    left = total_rows
    for i in range(n, 0, -1):
        even = (-(-left // i) + 7) // 8 * 8
        c = min(max_rows, even, left)
        sizes.append(c)
        left -= c
    return sizes


def kernel(x, pos_table):
    batch, seq_len = x.shape[0], x.shape[1]
    d_model = pos_table.shape[1]
    nw = _NC * _NS
    rows_per_w = seq_len // nw
    max_rows = min(rows_per_w, (131064 // d_model) // 8 * 8)
    sizes = _chunk_sizes(rows_per_w, max_rows)

    mesh = plsc.VectorSubcoreMesh(
        core_axis_name="c",
        subcore_axis_name="s",
        num_cores=_NC,
        num_subcores=_NS,
    )

    @functools.partial(
        pl.kernel,
        out_type=jax.ShapeDtypeStruct((batch, seq_len, d_model), jnp.float32),
        mesh=mesh,
        scratch_types=[
            pltpu.VMEM_SHARED((_NS, max_rows, d_model), jnp.float32),
            pltpu.SemaphoreType.DMA,
        ],
    )
    def broadcast_rows(table_hbm, out_hbm, shared, rsem):
        sid = lax.axis_index("s")
        wid = sid * _NC + lax.axis_index("c")
        base = wid * rows_per_w

        off = 0
        for c in sizes:
            r0 = base + off
            pltpu.async_copy(table_hbm.at[pl.ds(r0, c)],
                             shared.at[sid, pl.ds(0, c)], rsem).wait()
            for b in range(batch):
                pltpu.sync_copy(shared.at[sid, pl.ds(0, c)],
                                out_hbm.at[b, pl.ds(r0, c)])
            off += c

    return broadcast_rows(pos_table)


# speedup vs baseline: 1.1902x; 1.1902x over previous
import functools

import jax
import jax.numpy as jnp
from jax import lax
from jax.experimental import pallas as pl
from jax.experimental.pallas import tpu as pltpu
from jax.experimental.pallas import tpu_sc as plsc

_NC = 2
_NS = 16
_SP = 56


def _chunk_sizes(total_rows, max_rows):
    if total_rows <= 0:
        return []
    n = -(-total_rows // max_rows)
    sizes = []
    left = total_rows
    for i in range(n, 0, -1):
        even = (-(-left // i) + 7) // 8 * 8
        c = min(max_rows, even, left)
        sizes.append(c)
        left -= c
    return sizes


def kernel(x, pos_table):
    batch, seq_len = x.shape[0], x.shape[1]
    d_model = pos_table.shape[1]
    nw = _NC * _NS
    rows_per_w = seq_len // nw
    sp_rows = min(_SP, rows_per_w)
    tile_rows = rows_per_w - sp_rows
    max_rows = max(8, min(56 * 1024 // d_model * 8 // 8, (131064 // d_model) // 8 * 8))
    sizes = _chunk_sizes(tile_rows, max_rows)

    mesh = plsc.VectorSubcoreMesh(
        core_axis_name="c",
        subcore_axis_name="s",
        num_cores=_NC,
        num_subcores=_NS,
    )

    @functools.partial(
        pl.kernel,
        out_type=jax.ShapeDtypeStruct((batch, seq_len, d_model), jnp.float32),
        mesh=mesh,
        scratch_types=[
            pltpu.VMEM((min(max_rows, max(tile_rows, 8)), d_model), jnp.float32),
            pltpu.VMEM_SHARED((_NS, max(sp_rows, 8), d_model), jnp.float32),
            pltpu.SemaphoreType.DMA,
            pltpu.SemaphoreType.DMA,
            pltpu.SemaphoreType.DMA,
            pltpu.SemaphoreType.DMA,
        ],
    )
    def broadcast_rows(table_hbm, out_hbm, buf, shared, rsem_t, rsem_s,
                       wsem_t, wsem_s):
        sid = lax.axis_index("s")
        wid = sid * _NC + lax.axis_index("c")
        base = wid * rows_per_w
        sp0 = base + tile_rows

        if sp_rows:
            rd_s = pltpu.async_copy(table_hbm.at[pl.ds(sp0, sp_rows)],
                                    shared.at[sid, pl.ds(0, sp_rows)], rsem_s)

        off = 0
        pend = []
        for c in sizes:
            r0 = base + off
            for w in pend:
                w.wait()
            pltpu.async_copy(table_hbm.at[pl.ds(r0, c)],
                             buf.at[pl.ds(0, c)], rsem_t).wait()
            pend = [
                pltpu.async_copy(buf.at[pl.ds(0, c)],
                                 out_hbm.at[b, pl.ds(r0, c)], wsem_t)
                for b in range(batch)
            ]
            off += c

        if sp_rows:
            rd_s.wait()
            for b in range(batch):
                pltpu.sync_copy(shared.at[sid, pl.ds(0, sp_rows)],
                                out_hbm.at[b, pl.ds(sp0, sp_rows)])
        for w in pend:
            w.wait()

    return broadcast_rows(pos_table)
